# baseline (device time: 413737 ns/iter reference)
import jax
import jax.numpy as jnp
from jax import lax
from jax.experimental import pallas as pl
from jax.experimental.pallas import tpu as pltpu


def kernel(x):
    m, n = x.shape
    K = 16
    mc = m // K
    GROUPS = [(0, 2), (2, 2), (4, 4), (8, 8)]
    G = len(GROUPS)

    def body(x_ref, out_ref, xb_ref, f32_bufs, bf_bufs, ld_sems, ow_sems,
             xb_sems, send_sems, recv_sems):
        my_x = lax.axis_index("x")
        my_y = lax.axis_index("y")
        my_z = lax.axis_index("z")
        nbr = (my_x, 1 - my_y, my_z)

        barrier = pltpu.get_barrier_semaphore()
        pl.semaphore_signal(
            barrier, inc=1, device_id=nbr, device_id_type=pl.DeviceIdType.MESH
        )
        pl.semaphore_wait(barrier, 1)

        base = my_y * m

        def load(c):
            cp = pltpu.make_async_copy(
                x_ref.at[pl.ds(c * mc, mc), :],
                f32_bufs.at[c % 2],
                ld_sems.at[c % 2],
            )
            cp.start()
            return cp

        group_end = {s + cnt - 1: (g, s, cnt) for g, (s, cnt) in enumerate(GROUPS)}
        loads = [None] * K
        xb_sts = [None] * K
        owns = [None] * K
        rdmas = [None] * G
        loads[0] = load(0)
        for c in range(K):
            sl = c % 4
            if c + 1 < K:
                loads[c + 1] = load(c + 1)
            loads[c].wait()
            if c >= 4:
                owns[c - 4].wait()
                if xb_sts[c - 4] is not None:
                    xb_sts[c - 4].wait()
                    xb_sts[c - 4] = None
            bf_bufs[sl, :, :] = f32_bufs[c % 2, :, :].astype(jnp.bfloat16)
            xb_sts[c] = pltpu.make_async_copy(
                bf_bufs.at[sl], xb_ref.at[pl.ds(c * mc, mc), :], xb_sems.at[sl]
            )
            xb_sts[c].start()
            owns[c] = pltpu.make_async_copy(
                bf_bufs.at[sl],
                out_ref.at[pl.ds(base + c * mc, mc), :],
                ow_sems.at[sl],
            )
            owns[c].start()
            if c in group_end:
                g, s, cnt = group_end[c]
                for j in range(s, c + 1):
                    if xb_sts[j] is not None:
                        xb_sts[j].wait()
                        xb_sts[j] = None
                rdmas[g] = pltpu.make_async_remote_copy(
                    src_ref=xb_ref.at[pl.ds(s * mc, cnt * mc), :],
                    dst_ref=out_ref.at[pl.ds(base + s * mc, cnt * mc), :],
                    send_sem=send_sems.at[g],
                    recv_sem=recv_sems.at[g],
                    device_id=nbr,
                    device_id_type=pl.DeviceIdType.MESH,
                )
                rdmas[g].start()

        for c in range(K):
            if xb_sts[c] is not None:
                xb_sts[c].wait()
        for c in range(K - 4, K):
            owns[c].wait()
        for g in range(G):
            rdmas[g].wait()

    out, _ = pl.pallas_call(
        body,
        out_shape=[
            jax.ShapeDtypeStruct((2 * m, n), jnp.bfloat16),
            jax.ShapeDtypeStruct((m, n), jnp.bfloat16),
        ],
        in_specs=[pl.BlockSpec(memory_space=pl.ANY)],
        out_specs=[
            pl.BlockSpec(memory_space=pl.ANY),
            pl.BlockSpec(memory_space=pl.ANY),
        ],
        scratch_shapes=[
            pltpu.VMEM((2, mc, n), jnp.float32),
            pltpu.VMEM((4, mc, n), jnp.bfloat16),
            pltpu.SemaphoreType.DMA((2,)),
            pltpu.SemaphoreType.DMA((4,)),
            pltpu.SemaphoreType.DMA((4,)),
            pltpu.SemaphoreType.DMA((G,)),
            pltpu.SemaphoreType.DMA((G,)),
        ],
        compiler_params=pltpu.CompilerParams(collective_id=0),
    )(x)
    return out
